# Initial kernel scaffold; baseline (speedup 1.0000x reference)
import jax
import jax.numpy as jnp
from jax.experimental import pallas as pl


def kernel(logits):
    raise NotImplementedError("write your pallas kernel here")


# speedup vs baseline: 29.5208x; 29.5208x over previous
import jax
import jax.numpy as jnp
from jax import lax
from jax.experimental import pallas as pl

_RB = 8
_NEWTON_ITERS = 12


def _block_kernel(z_ref, probs_ref, act_ref, logp_ref, ent_ref, kl_ref):
    z = z_ref[...]
    rb, v = z.shape
    m = jnp.max(z, axis=1, keepdims=True)
    col = lax.broadcasted_iota(jnp.int32, z.shape, 1)
    am = jnp.min(jnp.where(z == m, col, v), axis=1, keepdims=True)

    zm = z - m
    e = jnp.exp(zm)
    s = jnp.sum(e, axis=1, keepdims=True)
    t = jnp.sum(zm * e, axis=1, keepdims=True)
    ent = jnp.log(s) - t / s

    def newton(_, tau):
        mask = z > tau
        k = jnp.sum(mask.astype(jnp.float32), axis=1, keepdims=True)
        ssel = jnp.sum(jnp.where(mask, z, 0.0), axis=1, keepdims=True)
        return jnp.maximum(tau, (ssel - 1.0) / k)

    tau = lax.fori_loop(0, _NEWTON_ITERS, newton, m - 1.0)

    probs_ref[...] = jnp.maximum(z - tau, 0.0)
    act_ref[...] = jnp.broadcast_to(am, (rb, 128)).astype(jnp.int32)
    logp_ref[...] = jnp.broadcast_to(jnp.log(1e-6 + (m - tau)), (rb, 128))
    ent_ref[...] = jnp.broadcast_to(ent, (rb, 128))
    kl_ref[...] = jnp.zeros((rb, 128), jnp.float32)


def kernel(logits):
    b, v = logits.shape
    grid = (b // _RB,)
    row_spec = pl.BlockSpec((_RB, v), lambda i: (i, 0))
    lane_spec = pl.BlockSpec((_RB, 128), lambda i: (i, 0))
    out_shape = [
        jax.ShapeDtypeStruct((b, v), jnp.float32),
        jax.ShapeDtypeStruct((b, 128), jnp.int32),
        jax.ShapeDtypeStruct((b, 128), jnp.float32),
        jax.ShapeDtypeStruct((b, 128), jnp.float32),
        jax.ShapeDtypeStruct((b, 128), jnp.float32),
    ]
    probs, act2, logp2, ent2, kl2 = pl.pallas_call(
        _block_kernel,
        grid=grid,
        in_specs=[row_spec],
        out_specs=[row_spec, lane_spec, lane_spec, lane_spec, lane_spec],
        out_shape=out_shape,
    )(logits)
    return (act2[:, 0], probs, logp2[:, 0], ent2[:, 0], kl2[:, 0])
